# baseline (speedup 1.0000x reference)
import jax
import jax.numpy as jnp
from jax.experimental import pallas as pl

_E_PAD = 128
_BT = 1024


def _gate_block(x, wt, bias):
    logits = jnp.dot(x, wt, preferred_element_type=jnp.float32) + bias
    m = jnp.max(logits, axis=1, keepdims=True)
    e = jnp.exp(logits - m)
    s = jnp.sum(e, axis=1, keepdims=True)
    sm = e / s
    v = jnp.max(sm, axis=1, keepdims=True)
    lane = jax.lax.broadcasted_iota(jnp.int32, sm.shape, 1)
    idx = jnp.min(jnp.where(sm >= v, lane, _E_PAD), axis=1, keepdims=True)
    return idx, v


def _gate_body(x0_ref, x1_ref, wt_ref, bias_ref,
               idx0_ref, score0_ref, idx1_ref, score1_ref):
    wt = wt_ref[...]
    bias = bias_ref[...]
    i0, v0 = _gate_block(x0_ref[...], wt, bias)
    idx0_ref[...] = i0
    score0_ref[...] = v0
    i1, v1 = _gate_block(x1_ref[...], wt, bias)
    idx1_ref[...] = i1
    score1_ref[...] = v1


def kernel(inp, W, b):
    T, D = inp.shape
    E = W.shape[0]
    wt = jnp.zeros((D, _E_PAD), dtype=jnp.float32).at[:, :E].set(W.T)
    bias = jnp.full((1, _E_PAD), -1e30, dtype=jnp.float32).at[0, :E].set(b)
    half = T // 2
    steps = half // _BT
    outs = pl.pallas_call(
        _gate_body,
        grid=(steps,),
        in_specs=[
            pl.BlockSpec((_BT, D), lambda i: (2 * i, 0)),
            pl.BlockSpec((_BT, D), lambda i: (2 * i + 1, 0)),
            pl.BlockSpec((D, _E_PAD), lambda i: (0, 0)),
            pl.BlockSpec((1, _E_PAD), lambda i: (0, 0)),
        ],
        out_specs=[
            pl.BlockSpec((_BT, 1), lambda i: (i, 0)),
            pl.BlockSpec((_BT, 1), lambda i: (i, 0)),
            pl.BlockSpec((_BT, 1), lambda i: (i, 0)),
            pl.BlockSpec((_BT, 1), lambda i: (i, 0)),
        ],
        out_shape=[
            jax.ShapeDtypeStruct((half, 1), jnp.int32),
            jax.ShapeDtypeStruct((half, 1), jnp.float32),
            jax.ShapeDtypeStruct((half, 1), jnp.int32),
            jax.ShapeDtypeStruct((half, 1), jnp.float32),
        ],
    )(inp, inp, wt, bias)
    idx0, score0, idx1, score1 = outs
    idx = jnp.stack([idx0.reshape(steps, _BT), idx1.reshape(steps, _BT)],
                    axis=1).reshape(T, 1)
    score = jnp.stack([score0.reshape(steps, _BT), score1.reshape(steps, _BT)],
                      axis=1).reshape(T, 1)
    return (idx.astype(jnp.int64), score)


# speedup vs baseline: 1.0516x; 1.0516x over previous
import jax
import jax.numpy as jnp
from jax.experimental import pallas as pl

_E_PAD = 128
_BT = 2048


def _gate_body(x_ref, wt_ref, bias_ref, idx_ref, score_ref):
    i = pl.program_id(0)
    x = x_ref[...]
    logits = jnp.dot(x, wt_ref[...], preferred_element_type=jnp.float32)
    logits = logits + bias_ref[...]
    m = jnp.max(logits, axis=1, keepdims=True)
    e = jnp.exp(logits - m)
    s = jnp.sum(e, axis=1, keepdims=True)
    sm = e / s
    v = jnp.max(sm, axis=1, keepdims=True)
    lane = jax.lax.broadcasted_iota(jnp.int32, sm.shape, 1)
    idx = jnp.min(jnp.where(sm >= v, lane, _E_PAD), axis=1, keepdims=True)
    idx_ref[pl.ds(i * _BT, _BT), :] = idx
    score_ref[pl.ds(i * _BT, _BT), :] = v


def kernel(inp, W, b):
    T, D = inp.shape
    E = W.shape[0]
    wt = jnp.zeros((D, _E_PAD), dtype=jnp.float32).at[:, :E].set(W.T)
    bias = jnp.full((1, _E_PAD), -1e30, dtype=jnp.float32).at[0, :E].set(b)
    grid = (T // _BT,)
    idx, score = pl.pallas_call(
        _gate_body,
        grid=grid,
        in_specs=[
            pl.BlockSpec((_BT, D), lambda i: (i, 0)),
            pl.BlockSpec((D, _E_PAD), lambda i: (0, 0)),
            pl.BlockSpec((1, _E_PAD), lambda i: (0, 0)),
        ],
        out_specs=[
            pl.BlockSpec((T, 1), lambda i: (0, 0)),
            pl.BlockSpec((T, 1), lambda i: (0, 0)),
        ],
        out_shape=[
            jax.ShapeDtypeStruct((T, 1), jnp.int32),
            jax.ShapeDtypeStruct((T, 1), jnp.float32),
        ],
    )(inp, wt, bias)
    return (idx.astype(jnp.int64), score)


# speedup vs baseline: 1.2021x; 1.1431x over previous
import jax
import jax.numpy as jnp
from jax.experimental import pallas as pl

_BT = 2048


def _gate_body(x_ref, wt_ref, bias_ref, idx_ref, score_ref):
    x = x_ref[...]
    logits = jnp.dot(x, wt_ref[...], preferred_element_type=jnp.float32)
    logits = logits + bias_ref[...]
    m = jnp.max(logits, axis=1, keepdims=True)
    e = jnp.exp(logits - m)
    s = jnp.sum(e, axis=1, keepdims=True)
    sm = e / s
    v = jnp.max(sm, axis=1, keepdims=True)
    lane = jax.lax.broadcasted_iota(jnp.int32, sm.shape, 1)
    idx = jnp.min(jnp.where(sm >= v, lane, sm.shape[1]), axis=1, keepdims=True)
    idx_ref[...] = idx
    score_ref[...] = v


def kernel(inp, W, b):
    T, D = inp.shape
    E = W.shape[0]
    wt = W.T
    bias = b.reshape(1, E)
    grid = (T // _BT,)
    idx, score = pl.pallas_call(
        _gate_body,
        grid=grid,
        in_specs=[
            pl.BlockSpec((_BT, D), lambda i: (i, 0)),
            pl.BlockSpec((D, E), lambda i: (0, 0)),
            pl.BlockSpec((1, E), lambda i: (0, 0)),
        ],
        out_specs=[
            pl.BlockSpec((_BT, 1), lambda i: (i, 0)),
            pl.BlockSpec((_BT, 1), lambda i: (i, 0)),
        ],
        out_shape=[
            jax.ShapeDtypeStruct((T, 1), jnp.int32),
            jax.ShapeDtypeStruct((T, 1), jnp.float32),
        ],
    )(inp, wt, bias)
    return (idx.astype(jnp.int64), score)
